# baseline (speedup 1.0000x reference)
import math

import jax
import jax.numpy as jnp
from jax.experimental import pallas as pl
from jax.experimental.pallas import tpu as pltpu

_B = 1024
_V = 100000
_LS = 0.1
_PAD = 0
_CONF = 1.0 - _LS
_SMOOTH = _LS / (_V - 2)
_CONST = _LS * math.log(_SMOOTH) + _CONF * math.log(_CONF)

_NS = 2
_BB = 32
_STEPS = _B // _NS // _BB


def _process(tgt_ref, x_ref, loss_ref, row0):
    x = x_ref[...]
    row_sum = jnp.sum(x, axis=1, keepdims=True)

    lane = jax.lax.broadcasted_iota(jnp.int32, (1, 128), 1)
    sel_rows = []
    t_rows = []
    for r in range(_BB):
        t_r = tgt_ref[row0 + r]
        base = (t_r // 128) * 128
        chunk = x_ref[r:r + 1, pl.ds(base, 128)]
        sel_rows.append(jnp.where(lane == t_r - base, chunk, 0.0))
        t_rows.append(jnp.full((1, 1), t_r, dtype=jnp.int32))
    o_t = jnp.sum(jnp.concatenate(sel_rows, axis=0), axis=1, keepdims=True)
    t_vec = jnp.concatenate(t_rows, axis=0)

    o_pad = x[:, _PAD:_PAD + 1]
    loss = _SMOOTH * row_sum + (_CONF - _SMOOTH) * o_t - _SMOOTH * o_pad - _CONST
    loss_ref[...] = jnp.where(t_vec == _PAD, 0.0, loss)


def _loss_kernel(tgt_ref, *refs):
    xs = refs[:_NS]
    outs = refs[_NS:]
    i = pl.program_id(0)
    for s in range(_NS):
        _process(tgt_ref, xs[s], outs[s], (i + s * _STEPS) * _BB)


def kernel(output, target):
    tgt = target.astype(jnp.int32)
    in_specs = [
        pl.BlockSpec((_BB, _V), (lambda i, t, s=s: (i + s * _STEPS, 0)))
        for s in range(_NS)
    ]
    out_specs = [
        pl.BlockSpec((_BB, 1), lambda i, t: (i, 0)) for _ in range(_NS)
    ]
    grid_spec = pltpu.PrefetchScalarGridSpec(
        num_scalar_prefetch=1,
        grid=(_STEPS,),
        in_specs=in_specs,
        out_specs=out_specs,
    )
    parts = pl.pallas_call(
        _loss_kernel,
        grid_spec=grid_spec,
        out_shape=[
            jax.ShapeDtypeStruct((_B // _NS, 1), jnp.float32)
            for _ in range(_NS)
        ],
    )(tgt, *([output] * _NS))
    return jnp.concatenate([p.reshape(-1) for p in parts])
